# baseline (speedup 1.0000x reference)
import functools

import numpy as np
import jax
import jax.numpy as jnp
from jax import lax
from jax.experimental import pallas as pl
from jax.experimental.pallas import tpu as pltpu
from jax.experimental.pallas import tpu_sc as plsc

_S = 8192
_W = 512
_NSINK = 4
_NCAS = _S // _W

_CHUNK = 128
_ZROWS = 512
_NBUF = 3


def _cascade_layout(T):
    cache = [-1] * _S
    start = [0] * _NCAS
    stored = [0] * _NCAS
    do_every = [2 ** i for i in range(_NCAS)]
    sink_ids = []
    seen = 0
    for t in range(T):
        seen += 1
        if len(sink_ids) < _NSINK:
            sink_ids.append(t)
            continue
        do_cache = [(seen - 1 - _NSINK) % do_every[i] == 0 for i in range(_NCAS)]
        tok = t
        ci = 0
        while tok is not None and ci < _NCAS:
            l = _W * ci
            if do_cache[ci]:
                if stored[ci] < _W:
                    cache[l + (start[ci] + stored[ci]) % _W] = tok
                    stored[ci] += 1
                    tok = None
                else:
                    s = l + start[ci]
                    evicted = cache[s]
                    cache[s] = tok
                    start[ci] = (start[ci] + 1) % _W
                    tok = evicted
                    ci += 1
            else:
                if stored[ci] > 0:
                    s = l + (start[ci] + stored[ci] - 1) % _W
                    cache[s] = tok
                tok = None
    slots = [i for i, v in enumerate(cache) if v >= 0]
    toks = [cache[i] for i in slots]
    return (np.asarray(sink_ids, np.int64), np.asarray(slots, np.int64),
            np.asarray(toks, np.int64))


@functools.lru_cache(maxsize=None)
def _gather_plan(T, H):
    sink_ids, slots, toks = _cascade_layout(T)
    dst = np.concatenate([np.arange(_NSINK), _NSINK + slots])
    src = np.concatenate([sink_ids, toks])
    order = np.argsort(dst, kind="stable")
    dst, src = dst[order], src[order]
    n = len(dst)
    assert np.array_equal(dst, np.arange(n)), "occupied slots not contiguous"

    reg = _NSINK + _S
    fpad = reg % 8
    bpad = (-n) % 8
    assert (n + bpad) % 8 == 0 and (fpad + n) % 8 == 0

    ent_k = list(src) + [None] * bpad
    ent_v = [None] * fpad + list(src)
    assert len(ent_k) == len(ent_v)
    ne = len(ent_k)
    nchunk = -(-ne // _CHUNK)
    last_m = ne - (nchunk - 1) * _CHUNK
    pad_total = nchunk * _CHUNK - ne

    origin = (0, reg - fpad)
    cover_end = (ne, reg - fpad + ne)

    idx = np.zeros((2 * H, nchunk, _CHUNK), np.int32)
    vzero = ([], [])
    for part, ents in enumerate((ent_k, ent_v)):
        full = ents + [None] * pad_total
        for j in range(nchunk):
            for r in range(_CHUNK):
                e = full[j * _CHUNK + r]
                if e is None and j * _CHUNK + r < ne:
                    vzero[part].append((j, r))
        base_idx = np.asarray([0 if e is None else e for e in full], np.int64)
        for h in range(H):
            idx[part * H + h] = (base_idx + h * T).astype(np.int32).reshape(
                nchunk, _CHUNK)

    zruns = ([], [])
    zend = (reg - fpad, 2 * reg)
    for part in range(2):
        z = cover_end[part]
        while z < zend[part]:
            m = min(_ZROWS, zend[part] - z)
            zruns[part].append((z, m))
            z += m
        assert cover_end[part] % 8 == 0

    return dict(idx=idx, nchunk=nchunk, last_m=last_m, origin=origin,
                vzero=vzero, zruns=zruns)


@functools.lru_cache(maxsize=None)
def _build_kernel(T, H, D):
    plan = _gather_plan(T, H)
    reg = _NSINK + _S
    outt = 2 * reg
    nunits = 2 * H
    assert nunits == 32, "one unit per TEC vector subcore"
    nchunk = plan["nchunk"]
    last_m = plan["last_m"]

    mesh = plsc.VectorSubcoreMesh(core_axis_name="c", subcore_axis_name="s")

    @functools.partial(
        pl.kernel,
        out_type=jax.ShapeDtypeStruct((H * outt, D), jnp.float32),
        mesh=mesh,
        scratch_types=(
            [pltpu.VMEM((nchunk, _CHUNK), jnp.int32)]
            + [pltpu.VMEM((_CHUNK, D), jnp.float32)] * _NBUF
            + [pltpu.VMEM((_ZROWS, D), jnp.float32)]
            + [pltpu.SemaphoreType.DMA] * (2 * _NBUF + 3)
        ),
    )
    def cache_fill(key_hbm, val_hbm, idx_hbm, zeros_hbm, out_hbm,
                   idx_v, *scratch):
        rows = scratch[:_NBUF]
        zbuf = scratch[_NBUF]
        gsems = scratch[_NBUF + 1:2 * _NBUF + 1]
        ssems = scratch[2 * _NBUF + 1:3 * _NBUF + 1]
        zsem, isem, zbsem = scratch[3 * _NBUF + 1:3 * _NBUF + 4]
        c = lax.axis_index("c")
        s = lax.axis_index("s")
        part = c
        head = s
        u = part * H + head
        hbase = head * outt

        idx_d = pltpu.async_copy(idx_hbm.at[u], idx_v, isem)
        zb_d = pltpu.async_copy(zeros_hbm, zbuf, zbsem)

        zero16 = jnp.zeros((16,), jnp.float32)

        def run(table, part_i):
            origin = plan["origin"][part_i]
            vzero = plan["vzero"][part_i]

            def fire_gather(j):
                return pltpu.async_copy(table.at[idx_v.at[j]],
                                        rows[j % _NBUF], gsems[j % _NBUF])

            def fire_store(j):
                m = _CHUNK if j < nchunk - 1 else last_m
                return pltpu.async_copy(
                    rows[j % _NBUF].at[pl.ds(0, m)],
                    out_hbm.at[pl.ds(hbase + origin + j * _CHUNK, m)],
                    ssems[j % _NBUF])

            gd = [None] * nchunk
            sd = [None] * nchunk
            idx_d.wait()
            for j in range(min(_NBUF - 1, nchunk)):
                gd[j] = fire_gather(j)
            zb_d.wait()
            zdescs = [
                pltpu.async_copy(zbuf.at[pl.ds(0, m)],
                                 out_hbm.at[pl.ds(hbase + z, m)], zsem)
                for (z, m) in plan["zruns"][part_i]
            ]
            for j in range(nchunk):
                gd[j].wait()
                for (jj, r) in vzero:
                    if jj == j:
                        for k in range(D // 16):
                            rows[j % _NBUF][r, pl.ds(k * 16, 16)] = zero16
                sd[j] = fire_store(j)
                nxt = j + _NBUF - 1
                if nxt < nchunk and gd[nxt] is None:
                    prev = nxt - _NBUF
                    if prev >= 0:
                        sd[prev].wait()
                        sd[prev] = None
                    gd[nxt] = fire_gather(nxt)
            for j in range(nchunk):
                if sd[j] is not None:
                    sd[j].wait()
            for d in zdescs:
                d.wait()

        @pl.when(part == 0)
        def _():
            run(key_hbm, 0)

        @pl.when(part == 1)
        def _():
            run(val_hbm, 1)

    return cache_fill


def kernel(key_states, value_states, layer_idx):
    del layer_idx
    B, H, T, D = key_states.shape
    assert B == 1
    plan = _gather_plan(T, H)
    fn = _build_kernel(T, H, D)
    out_flat = fn(
        key_states.reshape(H * T, D),
        value_states.reshape(H * T, D),
        jnp.asarray(plan["idx"]),
        jnp.zeros((_ZROWS, D), jnp.float32),
    )
    outt = 2 * (_NSINK + _S)
    return out_flat.reshape(B, H, outt, D)


# speedup vs baseline: 1.0209x; 1.0209x over previous
import functools

import numpy as np
import jax
import jax.numpy as jnp
from jax import lax
from jax.experimental import pallas as pl
from jax.experimental.pallas import tpu as pltpu
from jax.experimental.pallas import tpu_sc as plsc

_S = 8192
_W = 512
_NSINK = 4
_NCAS = _S // _W

_CHUNK = 128
_ZROWS = 128
_NBUF = 6


def _cascade_layout(T):
    cache = [-1] * _S
    start = [0] * _NCAS
    stored = [0] * _NCAS
    do_every = [2 ** i for i in range(_NCAS)]
    sink_ids = []
    seen = 0
    for t in range(T):
        seen += 1
        if len(sink_ids) < _NSINK:
            sink_ids.append(t)
            continue
        do_cache = [(seen - 1 - _NSINK) % do_every[i] == 0 for i in range(_NCAS)]
        tok = t
        ci = 0
        while tok is not None and ci < _NCAS:
            l = _W * ci
            if do_cache[ci]:
                if stored[ci] < _W:
                    cache[l + (start[ci] + stored[ci]) % _W] = tok
                    stored[ci] += 1
                    tok = None
                else:
                    s = l + start[ci]
                    evicted = cache[s]
                    cache[s] = tok
                    start[ci] = (start[ci] + 1) % _W
                    tok = evicted
                    ci += 1
            else:
                if stored[ci] > 0:
                    s = l + (start[ci] + stored[ci] - 1) % _W
                    cache[s] = tok
                tok = None
    slots = [i for i, v in enumerate(cache) if v >= 0]
    toks = [cache[i] for i in slots]
    return (np.asarray(sink_ids, np.int64), np.asarray(slots, np.int64),
            np.asarray(toks, np.int64))


@functools.lru_cache(maxsize=None)
def _gather_plan(T, H):
    sink_ids, slots, toks = _cascade_layout(T)
    dst = np.concatenate([np.arange(_NSINK), _NSINK + slots])
    src = np.concatenate([sink_ids, toks])
    order = np.argsort(dst, kind="stable")
    dst, src = dst[order], src[order]
    n = len(dst)
    assert np.array_equal(dst, np.arange(n)), "occupied slots not contiguous"

    reg = _NSINK + _S
    fpad = reg % 8
    bpad = (-n) % 8
    assert (n + bpad) % 8 == 0 and (fpad + n) % 8 == 0

    ent_k = list(src) + [None] * bpad
    ent_v = [None] * fpad + list(src)
    assert len(ent_k) == len(ent_v)
    ne = len(ent_k)
    nchunk = -(-ne // _CHUNK)
    last_m = ne - (nchunk - 1) * _CHUNK
    pad_total = nchunk * _CHUNK - ne

    origin = (0, reg - fpad)
    cover_end = (ne, reg - fpad + ne)

    idx = np.zeros((2 * H, nchunk, _CHUNK), np.int32)
    vzero = ([], [])
    for part, ents in enumerate((ent_k, ent_v)):
        full = ents + [None] * pad_total
        for j in range(nchunk):
            for r in range(_CHUNK):
                e = full[j * _CHUNK + r]
                if e is None and j * _CHUNK + r < ne:
                    vzero[part].append((j, r))
        base_idx = np.asarray([0 if e is None else e for e in full], np.int64)
        for h in range(H):
            idx[part * H + h] = (base_idx + h * T).astype(np.int32).reshape(
                nchunk, _CHUNK)

    zruns = ([], [])
    zend = (reg - fpad, 2 * reg)
    for part in range(2):
        z = cover_end[part]
        while z < zend[part]:
            m = min(_ZROWS, zend[part] - z)
            zruns[part].append((z, m))
            z += m
        assert cover_end[part] % 8 == 0

    return dict(idx=idx, nchunk=nchunk, last_m=last_m, origin=origin,
                vzero=vzero, zruns=zruns)


@functools.lru_cache(maxsize=None)
def _build_kernel(T, H, D):
    plan = _gather_plan(T, H)
    reg = _NSINK + _S
    outt = 2 * reg
    nunits = 2 * H
    assert nunits == 32, "one unit per TEC vector subcore"
    nchunk = plan["nchunk"]
    last_m = plan["last_m"]

    mesh = plsc.VectorSubcoreMesh(core_axis_name="c", subcore_axis_name="s")

    @functools.partial(
        pl.kernel,
        out_type=jax.ShapeDtypeStruct((H * outt, D), jnp.float32),
        mesh=mesh,
        scratch_types=(
            [pltpu.VMEM((nchunk, _CHUNK), jnp.int32)]
            + [pltpu.VMEM((_CHUNK, D), jnp.float32)] * _NBUF
            + [pltpu.VMEM((_ZROWS, D), jnp.float32)]
            + [pltpu.SemaphoreType.DMA] * (2 * _NBUF + 3)
        ),
    )
    def cache_fill(key_hbm, val_hbm, idx_hbm, zeros_hbm, out_hbm,
                   idx_v, *scratch):
        rows = scratch[:_NBUF]
        zbuf = scratch[_NBUF]
        gsems = scratch[_NBUF + 1:2 * _NBUF + 1]
        ssems = scratch[2 * _NBUF + 1:3 * _NBUF + 1]
        zsem, isem, zbsem = scratch[3 * _NBUF + 1:3 * _NBUF + 4]
        c = lax.axis_index("c")
        s = lax.axis_index("s")
        part = c
        head = s
        u = part * H + head
        hbase = head * outt

        idx_d = pltpu.async_copy(idx_hbm.at[u], idx_v, isem)
        zb_d = pltpu.async_copy(zeros_hbm, zbuf, zbsem)

        zero16 = jnp.zeros((16,), jnp.float32)

        def run(table, part_i):
            origin = plan["origin"][part_i]
            vzero = plan["vzero"][part_i]

            def fire_gather(j):
                return pltpu.async_copy(table.at[idx_v.at[j]],
                                        rows[j % _NBUF], gsems[j % _NBUF])

            def fire_store(j):
                m = _CHUNK if j < nchunk - 1 else last_m
                return pltpu.async_copy(
                    rows[j % _NBUF].at[pl.ds(0, m)],
                    out_hbm.at[pl.ds(hbase + origin + j * _CHUNK, m)],
                    ssems[j % _NBUF])

            gd = [None] * nchunk
            sd = [None] * nchunk
            idx_d.wait()
            for j in range(min(_NBUF - 1, nchunk)):
                gd[j] = fire_gather(j)
            zb_d.wait()
            zruns = plan["zruns"][part_i]
            per_iter = -(-len(zruns) // nchunk)
            zdescs = []

            def fire_zeros(j):
                for (z, m) in zruns[j * per_iter:(j + 1) * per_iter]:
                    zdescs.append(
                        pltpu.async_copy(zbuf.at[pl.ds(0, m)],
                                         out_hbm.at[pl.ds(hbase + z, m)],
                                         zsem))
            for j in range(nchunk):
                fire_zeros(j)
                gd[j].wait()
                for (jj, r) in vzero:
                    if jj == j:
                        for k in range(D // 16):
                            rows[j % _NBUF][r, pl.ds(k * 16, 16)] = zero16
                sd[j] = fire_store(j)
                nxt = j + _NBUF - 1
                if nxt < nchunk and gd[nxt] is None:
                    prev = nxt - _NBUF
                    if prev >= 0:
                        sd[prev].wait()
                        sd[prev] = None
                    gd[nxt] = fire_gather(nxt)
            for j in range(nchunk):
                if sd[j] is not None:
                    sd[j].wait()
            for d in zdescs:
                d.wait()

        @pl.when(part == 0)
        def _():
            run(key_hbm, 0)

        @pl.when(part == 1)
        def _():
            run(val_hbm, 1)

    return cache_fill


def kernel(key_states, value_states, layer_idx):
    del layer_idx
    B, H, T, D = key_states.shape
    assert B == 1
    plan = _gather_plan(T, H)
    fn = _build_kernel(T, H, D)
    out_flat = fn(
        key_states.reshape(H * T, D),
        value_states.reshape(H * T, D),
        jnp.asarray(plan["idx"]),
        jnp.zeros((_ZROWS, D), jnp.float32),
    )
    outt = 2 * (_NSINK + _S)
    return out_flat.reshape(B, H, outt, D)


# speedup vs baseline: 1.1328x; 1.1096x over previous
import functools

import numpy as np
import jax
import jax.numpy as jnp
from jax import lax
from jax.experimental import pallas as pl
from jax.experimental.pallas import tpu as pltpu
from jax.experimental.pallas import tpu_sc as plsc

_S = 8192
_W = 512
_NSINK = 4
_NCAS = _S // _W

_CHUNK = 128
_NBUF = 7
_ZBLK = 2048
_NZSEM = 8


def _cascade_layout(T):
    cache = [-1] * _S
    start = [0] * _NCAS
    stored = [0] * _NCAS
    do_every = [2 ** i for i in range(_NCAS)]
    sink_ids = []
    seen = 0
    for t in range(T):
        seen += 1
        if len(sink_ids) < _NSINK:
            sink_ids.append(t)
            continue
        do_cache = [(seen - 1 - _NSINK) % do_every[i] == 0 for i in range(_NCAS)]
        tok = t
        ci = 0
        while tok is not None and ci < _NCAS:
            l = _W * ci
            if do_cache[ci]:
                if stored[ci] < _W:
                    cache[l + (start[ci] + stored[ci]) % _W] = tok
                    stored[ci] += 1
                    tok = None
                else:
                    s = l + start[ci]
                    evicted = cache[s]
                    cache[s] = tok
                    start[ci] = (start[ci] + 1) % _W
                    tok = evicted
                    ci += 1
            else:
                if stored[ci] > 0:
                    s = l + (start[ci] + stored[ci] - 1) % _W
                    cache[s] = tok
                tok = None
    slots = [i for i, v in enumerate(cache) if v >= 0]
    toks = [cache[i] for i in slots]
    return (np.asarray(sink_ids, np.int64), np.asarray(slots, np.int64),
            np.asarray(toks, np.int64))


@functools.lru_cache(maxsize=None)
def _gather_plan(T, H):
    sink_ids, slots, toks = _cascade_layout(T)
    dst = np.concatenate([np.arange(_NSINK), _NSINK + slots])
    src = np.concatenate([sink_ids, toks])
    order = np.argsort(dst, kind="stable")
    dst, src = dst[order], src[order]
    n = len(dst)
    assert np.array_equal(dst, np.arange(n)), "occupied slots not contiguous"

    reg = _NSINK + _S
    outt = 2 * reg
    fpad = reg % 8
    bpad = (-n) % 8

    ent_k = list(src) + [None] * bpad
    ent_v = [None] * fpad + list(src)
    assert len(ent_k) == len(ent_v)
    ne = len(ent_k)
    nchunk = -(-ne // _CHUNK)
    last_m = ne - (nchunk - 1) * _CHUNK
    pad_total = nchunk * _CHUNK - ne

    origin = (0, reg - fpad)
    assert all(o % 8 == 0 and (o + ne) % 8 == 0 for o in origin)

    idx = np.zeros((2 * H, nchunk, _CHUNK), np.int32)
    vzero = ([], [])
    for part, ents in enumerate((ent_k, ent_v)):
        full = ents + [None] * pad_total
        for j in range(nchunk):
            for r in range(_CHUNK):
                e = full[j * _CHUNK + r]
                if e is None and j * _CHUNK + r < ne:
                    vzero[part].append((j, r))
        base_idx = np.asarray([0 if e is None else e for e in full], np.int64)
        for h in range(H):
            idx[part * H + h] = (base_idx + h * T).astype(np.int32).reshape(
                nchunk, _CHUNK)

    zruns = []
    spans = [(origin[0] + ne, origin[1]), (origin[1] + ne, outt)]
    for h in range(H):
        for (z0, z1) in spans:
            z = h * outt + z0
            end = h * outt + z1
            while z < end:
                m = min(_ZBLK, end - z)
                zruns.append((z, m))
                z += m
    return dict(idx=idx, nchunk=nchunk, last_m=last_m, origin=origin,
                vzero=vzero, zruns=zruns)


@functools.lru_cache(maxsize=None)
def _build_sc_gather(T, H, D):
    plan = _gather_plan(T, H)
    reg = _NSINK + _S
    outt = 2 * reg
    assert 2 * H == 32, "one unit per TEC vector subcore"
    nchunk = plan["nchunk"]
    last_m = plan["last_m"]

    mesh = plsc.VectorSubcoreMesh(core_axis_name="c", subcore_axis_name="s")

    @functools.partial(
        pl.kernel,
        out_type=jax.ShapeDtypeStruct((H * outt, D), jnp.float32),
        mesh=mesh,
        scratch_types=(
            [pltpu.VMEM((nchunk, _CHUNK), jnp.int32)]
            + [pltpu.VMEM((_CHUNK, D), jnp.float32)] * _NBUF
            + [pltpu.SemaphoreType.DMA] * (2 * _NBUF + 1)
        ),
    )
    def sc_gather(key_hbm, val_hbm, idx_hbm, out_hbm, idx_v, *scratch):
        rows = scratch[:_NBUF]
        gsems = scratch[_NBUF:2 * _NBUF]
        ssems = scratch[2 * _NBUF:3 * _NBUF]
        isem = scratch[3 * _NBUF]
        c = lax.axis_index("c")
        s = lax.axis_index("s")
        part = c
        head = s
        u = part * H + head
        hbase = head * outt

        idx_d = pltpu.async_copy(idx_hbm.at[u], idx_v, isem)
        zero16 = jnp.zeros((16,), jnp.float32)

        def run(table, part_i):
            origin = plan["origin"][part_i]
            vzero = plan["vzero"][part_i]

            def fire_gather(j):
                return pltpu.async_copy(table.at[idx_v.at[j]],
                                        rows[j % _NBUF], gsems[j % _NBUF])

            def fire_store(j):
                m = _CHUNK if j < nchunk - 1 else last_m
                return pltpu.async_copy(
                    rows[j % _NBUF].at[pl.ds(0, m)],
                    out_hbm.at[pl.ds(hbase + origin + j * _CHUNK, m)],
                    ssems[j % _NBUF])

            gd = [None] * nchunk
            sd = [None] * nchunk
            idx_d.wait()
            for j in range(min(_NBUF - 1, nchunk)):
                gd[j] = fire_gather(j)
            for j in range(nchunk):
                gd[j].wait()
                for (jj, r) in vzero:
                    if jj == j:
                        for k in range(D // 16):
                            rows[j % _NBUF][r, pl.ds(k * 16, 16)] = zero16
                sd[j] = fire_store(j)
                nxt = j + _NBUF - 1
                if nxt < nchunk and gd[nxt] is None:
                    prev = nxt - _NBUF
                    if prev >= 0:
                        sd[prev].wait()
                        sd[prev] = None
                    gd[nxt] = fire_gather(nxt)
            for j in range(nchunk):
                if sd[j] is not None:
                    sd[j].wait()

        @pl.when(part == 0)
        def _():
            run(key_hbm, 0)

        @pl.when(part == 1)
        def _():
            run(val_hbm, 1)

    return sc_gather


@functools.lru_cache(maxsize=None)
def _build_zero_blast(T, H, D):
    plan = _gather_plan(T, H)
    zruns = plan["zruns"]
    outt = 2 * (_NSINK + _S)
    nrows = H * outt

    def body(x_ref, o_ref, zbuf, *sems):
        del x_ref
        zbuf[...] = jnp.zeros((_ZBLK, D), jnp.float32)
        descs = []
        for i, (z, m) in enumerate(zruns):
            descs.append(
                pltpu.async_copy(zbuf.at[pl.ds(0, m)],
                                 o_ref.at[pl.ds(z, m)], sems[i % _NZSEM]))
        for d in descs:
            d.wait()

    return pl.pallas_call(
        body,
        out_shape=jax.ShapeDtypeStruct((nrows, D), jnp.float32),
        in_specs=[pl.BlockSpec(memory_space=pl.ANY)],
        out_specs=pl.BlockSpec(memory_space=pl.ANY),
        scratch_shapes=(
            [pltpu.VMEM((_ZBLK, D), jnp.float32)]
            + [pltpu.SemaphoreType.DMA] * _NZSEM
        ),
        input_output_aliases={0: 0},
    )


def kernel(key_states, value_states, layer_idx):
    del layer_idx
    B, H, T, D = key_states.shape
    assert B == 1
    plan = _gather_plan(T, H)
    gathered = _build_sc_gather(T, H, D)(
        key_states.reshape(H * T, D),
        value_states.reshape(H * T, D),
        jnp.asarray(plan["idx"]),
    )
    out_flat = _build_zero_blast(T, H, D)(gathered)
    outt = 2 * (_NSINK + _S)
    return out_flat.reshape(B, H, outt, D)


# speedup vs baseline: 1.1979x; 1.0574x over previous
import functools

import numpy as np
import jax
import jax.numpy as jnp
from jax import lax
from jax.experimental import pallas as pl
from jax.experimental.pallas import tpu as pltpu
from jax.experimental.pallas import tpu_sc as plsc

_S = 8192
_W = 512
_NSINK = 4
_NCAS = _S // _W

_CHUNK = 64
_NBUF = 14
_ZBLK = 2048
_NZSEM = 8


def _cascade_layout(T):
    cache = [-1] * _S
    start = [0] * _NCAS
    stored = [0] * _NCAS
    do_every = [2 ** i for i in range(_NCAS)]
    sink_ids = []
    seen = 0
    for t in range(T):
        seen += 1
        if len(sink_ids) < _NSINK:
            sink_ids.append(t)
            continue
        do_cache = [(seen - 1 - _NSINK) % do_every[i] == 0 for i in range(_NCAS)]
        tok = t
        ci = 0
        while tok is not None and ci < _NCAS:
            l = _W * ci
            if do_cache[ci]:
                if stored[ci] < _W:
                    cache[l + (start[ci] + stored[ci]) % _W] = tok
                    stored[ci] += 1
                    tok = None
                else:
                    s = l + start[ci]
                    evicted = cache[s]
                    cache[s] = tok
                    start[ci] = (start[ci] + 1) % _W
                    tok = evicted
                    ci += 1
            else:
                if stored[ci] > 0:
                    s = l + (start[ci] + stored[ci] - 1) % _W
                    cache[s] = tok
                tok = None
    slots = [i for i, v in enumerate(cache) if v >= 0]
    toks = [cache[i] for i in slots]
    return (np.asarray(sink_ids, np.int64), np.asarray(slots, np.int64),
            np.asarray(toks, np.int64))


@functools.lru_cache(maxsize=None)
def _gather_plan(T, H):
    sink_ids, slots, toks = _cascade_layout(T)
    dst = np.concatenate([np.arange(_NSINK), _NSINK + slots])
    src = np.concatenate([sink_ids, toks])
    order = np.argsort(dst, kind="stable")
    dst, src = dst[order], src[order]
    n = len(dst)
    assert np.array_equal(dst, np.arange(n)), "occupied slots not contiguous"

    reg = _NSINK + _S
    outt = 2 * reg
    fpad = reg % 8
    bpad = (-n) % 8

    ent_k = list(src) + [None] * bpad
    ent_v = [None] * fpad + list(src)
    assert len(ent_k) == len(ent_v)
    ne = len(ent_k)
    nchunk = -(-ne // _CHUNK)
    last_m = ne - (nchunk - 1) * _CHUNK
    pad_total = nchunk * _CHUNK - ne

    origin = (0, reg - fpad)
    assert all(o % 8 == 0 and (o + ne) % 8 == 0 for o in origin)

    idx = np.zeros((2 * H, nchunk, _CHUNK), np.int32)
    vzero = ([], [])
    for part, ents in enumerate((ent_k, ent_v)):
        full = ents + [None] * pad_total
        for j in range(nchunk):
            for r in range(_CHUNK):
                e = full[j * _CHUNK + r]
                if e is None and j * _CHUNK + r < ne:
                    vzero[part].append((j, r))
        base_idx = np.asarray([0 if e is None else e for e in full], np.int64)
        for h in range(H):
            idx[part * H + h] = (base_idx + h * T).astype(np.int32).reshape(
                nchunk, _CHUNK)

    zruns = []
    spans = [(origin[0] + ne, origin[1]), (origin[1] + ne, outt)]
    for h in range(H):
        for (z0, z1) in spans:
            z = h * outt + z0
            end = h * outt + z1
            while z < end:
                m = min(_ZBLK, end - z)
                zruns.append((z, m))
                z += m
    return dict(idx=idx, nchunk=nchunk, last_m=last_m, origin=origin,
                vzero=vzero, zruns=zruns)


@functools.lru_cache(maxsize=None)
def _build_sc_gather(T, H, D):
    plan = _gather_plan(T, H)
    reg = _NSINK + _S
    outt = 2 * reg
    assert 2 * H == 32, "one unit per TEC vector subcore"
    nchunk = plan["nchunk"]
    last_m = plan["last_m"]

    mesh = plsc.VectorSubcoreMesh(core_axis_name="c", subcore_axis_name="s")

    @functools.partial(
        pl.kernel,
        out_type=jax.ShapeDtypeStruct((H * outt, D), jnp.float32),
        mesh=mesh,
        scratch_types=(
            [pltpu.VMEM((nchunk, _CHUNK), jnp.int32)]
            + [pltpu.VMEM((_CHUNK, D), jnp.float32)] * _NBUF
            + [pltpu.SemaphoreType.DMA] * (2 * _NBUF + 1)
        ),
    )
    def sc_gather(key_hbm, val_hbm, idx_hbm, out_hbm, idx_v, *scratch):
        rows = scratch[:_NBUF]
        gsems = scratch[_NBUF:2 * _NBUF]
        ssems = scratch[2 * _NBUF:3 * _NBUF]
        isem = scratch[3 * _NBUF]
        c = lax.axis_index("c")
        s = lax.axis_index("s")
        part = c
        head = s
        u = part * H + head
        hbase = head * outt

        idx_d = pltpu.async_copy(idx_hbm.at[u], idx_v, isem)
        zero16 = jnp.zeros((16,), jnp.float32)

        def run(table, part_i):
            origin = plan["origin"][part_i]
            vzero = plan["vzero"][part_i]

            def fire_gather(j):
                return pltpu.async_copy(table.at[idx_v.at[j]],
                                        rows[j % _NBUF], gsems[j % _NBUF])

            def fire_store(j):
                m = _CHUNK if j < nchunk - 1 else last_m
                return pltpu.async_copy(
                    rows[j % _NBUF].at[pl.ds(0, m)],
                    out_hbm.at[pl.ds(hbase + origin + j * _CHUNK, m)],
                    ssems[j % _NBUF])

            gd = [None] * nchunk
            sd = [None] * nchunk
            idx_d.wait()
            for j in range(min(_NBUF - 1, nchunk)):
                gd[j] = fire_gather(j)
            for j in range(nchunk):
                gd[j].wait()
                for (jj, r) in vzero:
                    if jj == j:
                        for k in range(D // 16):
                            rows[j % _NBUF][r, pl.ds(k * 16, 16)] = zero16
                sd[j] = fire_store(j)
                nxt = j + _NBUF - 1
                if nxt < nchunk and gd[nxt] is None:
                    prev = nxt - _NBUF
                    if prev >= 0:
                        sd[prev].wait()
                        sd[prev] = None
                    gd[nxt] = fire_gather(nxt)
            for j in range(nchunk):
                if sd[j] is not None:
                    sd[j].wait()

        @pl.when(part == 0)
        def _():
            run(key_hbm, 0)

        @pl.when(part == 1)
        def _():
            run(val_hbm, 1)

    return sc_gather


@functools.lru_cache(maxsize=None)
def _build_zero_blast(T, H, D):
    plan = _gather_plan(T, H)
    zruns = plan["zruns"]
    outt = 2 * (_NSINK + _S)
    nrows = H * outt

    def body(x_ref, o_ref, zbuf, *sems):
        del x_ref
        zbuf[...] = jnp.zeros((_ZBLK, D), jnp.float32)
        descs = []
        for i, (z, m) in enumerate(zruns):
            descs.append(
                pltpu.async_copy(zbuf.at[pl.ds(0, m)],
                                 o_ref.at[pl.ds(z, m)], sems[i % _NZSEM]))
        for d in descs:
            d.wait()

    return pl.pallas_call(
        body,
        out_shape=jax.ShapeDtypeStruct((nrows, D), jnp.float32),
        in_specs=[pl.BlockSpec(memory_space=pl.ANY)],
        out_specs=pl.BlockSpec(memory_space=pl.ANY),
        scratch_shapes=(
            [pltpu.VMEM((_ZBLK, D), jnp.float32)]
            + [pltpu.SemaphoreType.DMA] * _NZSEM
        ),
        input_output_aliases={0: 0},
    )


def kernel(key_states, value_states, layer_idx):
    del layer_idx
    B, H, T, D = key_states.shape
    assert B == 1
    plan = _gather_plan(T, H)
    gathered = _build_sc_gather(T, H, D)(
        key_states.reshape(H * T, D),
        value_states.reshape(H * T, D),
        jnp.asarray(plan["idx"]),
    )
    out_flat = _build_zero_blast(T, H, D)(gathered)
    outt = 2 * (_NSINK + _S)
    return out_flat.reshape(B, H, outt, D)
